# baseline (speedup 1.0000x reference)
import functools
import jax
import jax.numpy as jnp
from jax import lax
from jax.experimental import pallas as pl
from jax.experimental.pallas import tpu as pltpu
from jax.experimental.pallas import tpu_sc as plsc

BATCH = 16384
LATENT = 16
NC = 2
NS = 16
NW = NC * NS
BPW = BATCH // NW
CHUNK = 128
CHUNKS = BPW // CHUNK
GROUPS = BPW // LATENT


def _gmf_body(uidx_hbm, iidx_hbm, eu_hbm, ei_hbm, wmat_hbm, bvec_hbm, out_hbm,
              uidx_v, iidx_v, urows_v, irows_v, wmat_v, bvec_v, logits_v, sem):
    wid = lax.axis_index("s") * NC + lax.axis_index("c")

    pltpu.sync_copy(uidx_hbm.at[wid], uidx_v)
    pltpu.sync_copy(iidx_hbm.at[wid], iidx_v)
    pltpu.sync_copy(wmat_hbm, wmat_v)
    pltpu.sync_copy(bvec_hbm, bvec_v)

    copies = []
    for j in range(CHUNKS):
        copies.append(pltpu.async_copy(
            eu_hbm.at[uidx_v.at[j]], urows_v.at[pl.ds(j * CHUNK, CHUNK)], sem))
        copies.append(pltpu.async_copy(
            ei_hbm.at[iidx_v.at[j]], irows_v.at[pl.ds(j * CHUNK, CHUNK)], sem))
    for c in copies:
        c.wait()

    bv = bvec_v[...]
    iota = lax.iota(jnp.int32, LATENT)

    def group_body(g, carry):
        rows = g * LATENT + iota
        acc = bv
        for d in range(LATENT):
            dsplat = jnp.full((LATENT,), d, jnp.int32)
            u_col = plsc.load_gather(urows_v, [rows, dsplat])
            i_col = plsc.load_gather(irows_v, [rows, dsplat])
            acc = acc + u_col * i_col * wmat_v[d, :]
        logits_v[pl.ds(g * LATENT, LATENT)] = 1.0 / (1.0 + jnp.exp(-acc))
        return carry

    lax.fori_loop(0, GROUPS, group_body, 0)

    pltpu.sync_copy(logits_v, out_hbm.at[pl.ds(wid * BPW, BPW)])


def kernel(user_indices, item_indices, domain_idc, embedding_user,
           embedding_item, affine_W, affine_b):
    del domain_idc
    uidx = user_indices.astype(jnp.int32).reshape(NW, CHUNKS, CHUNK)
    iidx = item_indices.astype(jnp.int32).reshape(NW, CHUNKS, CHUNK)
    w = affine_W.reshape(LATENT)
    wmat = jnp.broadcast_to(w[:, None], (LATENT, LATENT))
    bvec = jnp.broadcast_to(affine_b.reshape(1), (LATENT,))

    mesh = plsc.VectorSubcoreMesh(core_axis_name="c", subcore_axis_name="s")
    run = functools.partial(
        pl.kernel,
        out_type=jax.ShapeDtypeStruct((BATCH,), jnp.float32),
        mesh=mesh,
        compiler_params=pltpu.CompilerParams(
            needs_layout_passes=False, use_tc_tiling_on_sc=False),
        scratch_types=[
            pltpu.VMEM((CHUNKS, CHUNK), jnp.int32),
            pltpu.VMEM((CHUNKS, CHUNK), jnp.int32),
            pltpu.VMEM((BPW, LATENT), jnp.float32),
            pltpu.VMEM((BPW, LATENT), jnp.float32),
            pltpu.VMEM((LATENT, LATENT), jnp.float32),
            pltpu.VMEM((LATENT,), jnp.float32),
            pltpu.VMEM((BPW,), jnp.float32),
            pltpu.SemaphoreType.DMA,
        ],
    )(_gmf_body)
    out = run(uidx, iidx, embedding_user, embedding_item, wmat, bvec)
    return out.reshape(BATCH, 1)
